# Initial kernel scaffold; baseline (speedup 1.0000x reference)
import jax
import jax.numpy as jnp
from jax.experimental import pallas as pl


def kernel(x_user, x_item, W1_follows, b1_follows, W1_clicks, b1_clicks, W1_cb, b1_cb, W2_follows, b2_follows, W2_clicks, b2_clicks, W2_cb, b2_cb, edge_follows, edge_clicks, edge_clickedby):
    raise NotImplementedError("write your pallas kernel here")


# speedup vs baseline: 5.0367x; 5.0367x over previous
import functools

import jax
import jax.numpy as jnp
from jax import lax
from jax.experimental import pallas as pl
from jax.experimental.pallas import tpu as pltpu
from jax.experimental.pallas import tpu_sc as plsc

N = 50000
E = 500000
IN_SIZE = 128
HIDDEN = 64
OUT_SIZE = 32

N_ACC = 51200
R_TILE = N_ACC // 16
CHUNK = 128
E_PAD = 503808
K_CHUNKS = E_PAD // 16 // CHUNK
DRAIN = R_TILE // CHUNK


def _sc_body(fw, dt, npass, *args):
    ins = args[:3 * npass]
    outs = args[3 * npass:5 * npass]
    (srcv, dstv, gbuf, gbufb, zbuf, onesv, dbuf, acc, deg,
     gsem, gsemb) = args[5 * npass:]

    c = lax.axis_index("c")
    s = lax.axis_index("s")
    zero16 = jnp.zeros((16,), jnp.float32)
    one16 = jnp.ones((16,), jnp.float32)
    lanes = 16 if dt == jnp.float32 else 32
    zerov = jnp.zeros((lanes,), dt)

    def _fill_z(r, _):
        for q in range(fw // lanes):
            zbuf[r, pl.ds(q * lanes, lanes)] = zerov
        return 0
    lax.fori_loop(0, CHUNK, _fill_z, 0)

    def _fill_o(i, _):
        onesv[pl.ds(i * 16, 16)] = one16
        return 0
    lax.fori_loop(0, CHUNK // 16, _fill_o, 0)

    def _fill_d(i, _):
        dbuf[pl.ds(i * 16, 16)] = zero16
        return 0
    lax.fori_loop(0, R_TILE // 16, _fill_d, 0)

    for p in range(npass):
        tbl, srci, dsti = ins[3 * p:3 * p + 3]
        out_acc, out_deg = outs[2 * p:2 * p + 2]
        deg_core = p % 2

        pltpu.sync_copy(srci.at[c, s], srcv)
        pltpu.sync_copy(dsti.at[s], dstv)

        def _zero_acc(j, _):
            pltpu.sync_copy(zbuf, acc.at[pl.ds(s * R_TILE + j * CHUNK, CHUNK)])
            return 0
        lax.fori_loop(0, DRAIN, _zero_acc, 0)

        @pl.when(c == deg_core)
        def _():
            pltpu.sync_copy(dbuf, deg.at[pl.ds(s * R_TILE, R_TILE)])

        plsc.subcore_barrier()

        pltpu.async_copy(tbl.at[srcv.at[0]], gbuf, gsem)

        def _edge(j2, _):
            j = 2 * j2
            pltpu.async_copy(tbl.at[srcv.at[j + 1]], gbufb, gsemb)
            pltpu.make_async_copy(tbl.at[srcv.at[j]], gbuf, gsem).wait()
            pltpu.sync_copy(gbuf, acc.at[dstv.at[j]], add=True)

            @pl.when(c == deg_core)
            def _():
                pltpu.sync_copy(onesv, deg.at[dstv.at[j]], add=True)

            @pl.when(j + 2 < K_CHUNKS)
            def _():
                pltpu.async_copy(tbl.at[srcv.at[j + 2]], gbuf, gsem)
            pltpu.make_async_copy(tbl.at[srcv.at[j + 1]], gbufb, gsemb).wait()
            pltpu.sync_copy(gbufb, acc.at[dstv.at[j + 1]], add=True)

            @pl.when(c == deg_core)
            def _():
                pltpu.sync_copy(onesv, deg.at[dstv.at[j + 1]], add=True)
            return 0
        lax.fori_loop(0, K_CHUNKS // 2, _edge, 0)

        plsc.subcore_barrier()

        def _drain(j, _):
            r0 = s * R_TILE + j * CHUNK
            pltpu.sync_copy(acc.at[pl.ds(r0, CHUNK)], gbuf)
            pltpu.sync_copy(gbuf, out_acc.at[c, pl.ds(r0, CHUNK)])
            return 0
        lax.fori_loop(0, DRAIN, _drain, 0)

        @pl.when(c == deg_core)
        def _():
            pltpu.sync_copy(deg.at[pl.ds(s * R_TILE, R_TILE)], dbuf)
            pltpu.sync_copy(dbuf, out_deg.at[pl.ds(s * R_TILE, R_TILE)])
            if p + 1 < npass:
                lax.fori_loop(0, R_TILE // 16, _fill_d, 0)


@functools.lru_cache(maxsize=None)
def _make_sc_pass(fw, dt, npass):
    mesh = plsc.VectorSubcoreMesh(core_axis_name="c", subcore_axis_name="s")
    return pl.kernel(
        functools.partial(_sc_body, fw, dt, npass),
        out_type=(jax.ShapeDtypeStruct((2, N_ACC, fw), dt),
                  jax.ShapeDtypeStruct((N_ACC,), jnp.float32)) * npass,
        mesh=mesh,
        scratch_types=[
            pltpu.VMEM((K_CHUNKS, CHUNK), jnp.int32),
            pltpu.VMEM((K_CHUNKS, CHUNK), jnp.int32),
            pltpu.VMEM((CHUNK, fw), dt),
            pltpu.VMEM((CHUNK, fw), dt),
            pltpu.VMEM((CHUNK, fw), dt),
            pltpu.VMEM((CHUNK,), jnp.float32),
            pltpu.VMEM((R_TILE,), jnp.float32),
            pltpu.VMEM_SHARED((N_ACC, fw), dt),
            pltpu.VMEM_SHARED((N_ACC,), jnp.float32),
            pltpu.SemaphoreType.DMA,
            pltpu.SemaphoreType.DMA,
        ],
        compiler_params=pltpu.CompilerParams(use_tc_tiling_on_sc=False),
        name=f"rgcn_seg_sum_{fw}x{npass}",
    )


def _prep_edges(edge, n_tbl):
    src = edge[0].astype(jnp.int32)
    dst = edge[1].astype(jnp.int32)
    pad = E_PAD - E
    ar = jnp.arange(pad, dtype=jnp.int32)
    src_p = jnp.concatenate([src, ar % N])
    dst_p = jnp.concatenate([dst, N + (ar % (N_ACC - N))])
    srci = jnp.stack([src_p, src_p + n_tbl]).reshape(2, 16, K_CHUNKS, CHUNK)
    dsti = dst_p.reshape(16, K_CHUNKS, CHUNK)
    return lax.optimization_barrier((srci, dsti))


def _lin_body(x_ref, w_ref, b_ref, o_ref):
    o_ref[...] = (jnp.dot(x_ref[...], w_ref[0],
                          preferred_element_type=jnp.float32)
                  + b_ref[0]).astype(jnp.bfloat16)


def _linear_split(x, w, b):
    bm = 400
    nb = N // bm
    w3 = w.reshape(IN_SIZE, 2, 32).transpose(1, 0, 2)
    b3 = b.reshape(2, 1, 32)
    return pl.pallas_call(
        _lin_body,
        grid=(2, nb),
        in_specs=[
            pl.BlockSpec((bm, IN_SIZE), lambda h, i: (i, 0)),
            pl.BlockSpec((1, IN_SIZE, 32), lambda h, i: (h, 0, 0)),
            pl.BlockSpec((1, 1, 32), lambda h, i: (h, 0, 0)),
        ],
        out_specs=pl.BlockSpec((bm, 32), lambda h, i: (h * nb + i, 0)),
        out_shape=jax.ShapeDtypeStruct((2 * N, 32), jnp.bfloat16),
    )(x, w3, b3)


def _mid_body(af_ref, df_ref, acb_ref, dcb_ref, w_ref, b_ref, o_ref):
    df = jnp.maximum(df_ref[...], 1.0)
    dcb = jnp.maximum(dcb_ref[...], 1.0)
    h_lo = (af_ref[0].astype(jnp.float32) / df
            + acb_ref[0].astype(jnp.float32) / dcb)
    h_hi = (af_ref[1].astype(jnp.float32) / df
            + acb_ref[1].astype(jnp.float32) / dcb)
    h = jnp.concatenate([h_lo, h_hi], axis=1)
    h = jnp.where(h >= 0, h, 0.01 * h)
    o_ref[...] = jnp.dot(h, w_ref[0],
                         preferred_element_type=jnp.float32) + b_ref[0]


def _mid_layer(acc_f, deg_f, acc_cb, deg_cb, w2, b2):
    bm = 512
    nb = N_ACC // bm
    w3 = w2.reshape(HIDDEN, 2, 16).transpose(1, 0, 2)
    b3 = b2.reshape(2, 1, 16)
    return pl.pallas_call(
        _mid_body,
        grid=(2, nb),
        in_specs=[
            pl.BlockSpec((2, bm, 32), lambda h, i: (0, i, 0)),
            pl.BlockSpec((bm, 1), lambda h, i: (i, 0)),
            pl.BlockSpec((2, bm, 32), lambda h, i: (0, i, 0)),
            pl.BlockSpec((bm, 1), lambda h, i: (i, 0)),
            pl.BlockSpec((1, HIDDEN, 16), lambda h, i: (h, 0, 0)),
            pl.BlockSpec((1, 1, 16), lambda h, i: (h, 0, 0)),
        ],
        out_specs=pl.BlockSpec((bm, 16), lambda h, i: (h * nb + i, 0)),
        out_shape=jax.ShapeDtypeStruct((2 * N_ACC, 16), jnp.float32),
    )(acc_f, deg_f, acc_cb, deg_cb, w3, b3)


def _out_body(a_ref, d_ref, o_ref):
    d = jnp.maximum(d_ref[...], 1.0)
    o_ref[...] = jnp.concatenate([a_ref[0] / d, a_ref[1] / d], axis=1)


def _out_layer(acc_c, deg_c):
    bm = 400
    nb = N // bm
    return pl.pallas_call(
        _out_body,
        grid=(nb,),
        in_specs=[
            pl.BlockSpec((2, bm, 16), lambda i: (0, i, 0)),
            pl.BlockSpec((bm, 1), lambda i: (i, 0)),
        ],
        out_specs=pl.BlockSpec((bm, OUT_SIZE), lambda i: (i, 0)),
        out_shape=jax.ShapeDtypeStruct((N, OUT_SIZE), jnp.float32),
    )(acc_c, deg_c)


def kernel(x_user, x_item,
           W1_follows, b1_follows, W1_clicks, b1_clicks, W1_cb, b1_cb,
           W2_follows, b2_follows, W2_clicks, b2_clicks, W2_cb, b2_cb,
           edge_follows, edge_clicks, edge_clickedby):
    tbl_f = _linear_split(x_user, W1_follows, b1_follows)
    tbl_cb = _linear_split(x_item, W1_cb, b1_cb)

    sc32x2 = _make_sc_pass(32, jnp.bfloat16, 2)
    acc_f, deg_f, acc_cb, deg_cb = sc32x2(
        tbl_f, *_prep_edges(edge_follows, N),
        tbl_cb, *_prep_edges(edge_clickedby, N))

    tbl_c = _mid_layer(acc_f, deg_f[:, None], acc_cb, deg_cb[:, None],
                       W2_clicks, b2_clicks)

    sc16 = _make_sc_pass(16, jnp.float32, 1)
    acc_c, deg_c = sc16(tbl_c, *_prep_edges(edge_clicks, N_ACC))

    return _out_layer(acc_c, deg_c[:N, None])


# speedup vs baseline: 6.6090x; 1.3122x over previous
import functools

import jax
import jax.numpy as jnp
from jax import lax
from jax.experimental import pallas as pl
from jax.experimental.pallas import tpu as pltpu
from jax.experimental.pallas import tpu_sc as plsc

N = 50000
E = 500000
IN_SIZE = 128
HIDDEN = 64
OUT_SIZE = 32

N_ACC = 51200
R_TILE = N_ACC // 16
CHUNK = 128
E_PAD = 503808
K_CHUNKS = E_PAD // 16 // CHUNK
DRAIN = R_TILE // CHUNK


def _sc_body(fw, dt, npass, *args):
    ins = args[:3 * npass]
    outs = args[3 * npass:5 * npass]
    (srcv, dstv, gbuf, gbufb, zbuf, onesv, dbuf, acc, deg,
     gsem, gsemb) = args[5 * npass:]

    c = lax.axis_index("c")
    s = lax.axis_index("s")
    zero16 = jnp.zeros((16,), jnp.float32)
    one16 = jnp.ones((16,), jnp.float32)
    lanes = 16 if dt == jnp.float32 else 32
    zerov = jnp.zeros((lanes,), dt)

    def _fill_z(r, _):
        for q in range(fw // lanes):
            zbuf[r, pl.ds(q * lanes, lanes)] = zerov
        return 0
    lax.fori_loop(0, CHUNK, _fill_z, 0)

    def _fill_o(i, _):
        onesv[pl.ds(i * 16, 16)] = one16
        return 0
    lax.fori_loop(0, CHUNK // 16, _fill_o, 0)

    def _fill_d(i, _):
        dbuf[pl.ds(i * 16, 16)] = zero16
        return 0
    lax.fori_loop(0, R_TILE // 16, _fill_d, 0)

    for p in range(npass):
        tbl, srci, dsti = ins[3 * p:3 * p + 3]
        out_acc, out_deg = outs[2 * p:2 * p + 2]
        deg_core = p % 2

        pltpu.sync_copy(srci.at[c, s], srcv)
        pltpu.sync_copy(dsti.at[s], dstv)

        def _zero_acc(j, _):
            pltpu.sync_copy(zbuf, acc.at[pl.ds(s * R_TILE + j * CHUNK, CHUNK)])
            return 0
        lax.fori_loop(0, DRAIN, _zero_acc, 0)

        @pl.when(c == deg_core)
        def _():
            pltpu.sync_copy(dbuf, deg.at[pl.ds(s * R_TILE, R_TILE)])

        plsc.subcore_barrier()

        pltpu.async_copy(tbl.at[srcv.at[0]], gbuf, gsem)

        def _edge(j2, _):
            j = 2 * j2
            pltpu.async_copy(tbl.at[srcv.at[j + 1]], gbufb, gsemb)
            pltpu.make_async_copy(tbl.at[srcv.at[j]], gbuf, gsem).wait()
            pltpu.sync_copy(gbuf, acc.at[dstv.at[j]], add=True)

            @pl.when(c == deg_core)
            def _():
                pltpu.sync_copy(onesv, deg.at[dstv.at[j]], add=True)

            @pl.when(j + 2 < K_CHUNKS)
            def _():
                pltpu.async_copy(tbl.at[srcv.at[j + 2]], gbuf, gsem)
            pltpu.make_async_copy(tbl.at[srcv.at[j + 1]], gbufb, gsemb).wait()
            pltpu.sync_copy(gbufb, acc.at[dstv.at[j + 1]], add=True)

            @pl.when(c == deg_core)
            def _():
                pltpu.sync_copy(onesv, deg.at[dstv.at[j + 1]], add=True)
            return 0
        lax.fori_loop(0, K_CHUNKS // 2, _edge, 0)

        plsc.subcore_barrier()

        def _drain(j, _):
            r0 = s * R_TILE + j * CHUNK
            pltpu.sync_copy(acc.at[pl.ds(r0, CHUNK)], gbuf)
            pltpu.sync_copy(gbuf, out_acc.at[c, pl.ds(r0, CHUNK)])
            return 0
        lax.fori_loop(0, DRAIN, _drain, 0)

        @pl.when(c == deg_core)
        def _():
            pltpu.sync_copy(deg.at[pl.ds(s * R_TILE, R_TILE)], dbuf)
            pltpu.sync_copy(dbuf, out_deg.at[pl.ds(s * R_TILE, R_TILE)])
            if p + 1 < npass:
                lax.fori_loop(0, R_TILE // 16, _fill_d, 0)


@functools.lru_cache(maxsize=None)
def _make_sc_pass(fw, dt, npass):
    mesh = plsc.VectorSubcoreMesh(core_axis_name="c", subcore_axis_name="s")
    return pl.kernel(
        functools.partial(_sc_body, fw, dt, npass),
        out_type=(jax.ShapeDtypeStruct((2, N_ACC, fw), dt),
                  jax.ShapeDtypeStruct((N_ACC,), jnp.float32)) * npass,
        mesh=mesh,
        scratch_types=[
            pltpu.VMEM((K_CHUNKS, CHUNK), jnp.int32),
            pltpu.VMEM((K_CHUNKS, CHUNK), jnp.int32),
            pltpu.VMEM((CHUNK, fw), dt),
            pltpu.VMEM((CHUNK, fw), dt),
            pltpu.VMEM((CHUNK, fw), dt),
            pltpu.VMEM((CHUNK,), jnp.float32),
            pltpu.VMEM((R_TILE,), jnp.float32),
            pltpu.VMEM_SHARED((N_ACC, fw), dt),
            pltpu.VMEM_SHARED((N_ACC,), jnp.float32),
            pltpu.SemaphoreType.DMA,
            pltpu.SemaphoreType.DMA,
        ],
        compiler_params=pltpu.CompilerParams(use_tc_tiling_on_sc=False),
        name=f"rgcn_seg_sum_{fw}x{npass}",
    )


def _prep_edges(edge, n_tbl):
    src = edge[0].astype(jnp.int32)
    dst = edge[1].astype(jnp.int32)
    pad = E_PAD - E
    ar = jnp.arange(pad, dtype=jnp.int32)
    src_p = jnp.concatenate([src, ar % N])
    dst_p = jnp.concatenate([dst, N + (ar % (N_ACC - N))])
    srci = jnp.stack([src_p, src_p + n_tbl]).reshape(2, 16, K_CHUNKS, CHUNK)
    dsti = dst_p.reshape(16, K_CHUNKS, CHUNK)
    return lax.optimization_barrier((srci, dsti))


def _lin_body(x_ref, w_ref, b_ref, o_ref):
    o_ref[...] = (jnp.dot(x_ref[...], w_ref[0],
                          preferred_element_type=jnp.float32)
                  + b_ref[0]).astype(jnp.bfloat16)


def _linear_split(x, w, b):
    bm = 2000
    nb = N // bm
    w3 = w.reshape(IN_SIZE, 2, 32).transpose(1, 0, 2)
    b3 = b.reshape(2, 1, 32)
    return pl.pallas_call(
        _lin_body,
        grid=(2, nb),
        in_specs=[
            pl.BlockSpec((bm, IN_SIZE), lambda h, i: (i, 0)),
            pl.BlockSpec((1, IN_SIZE, 32), lambda h, i: (h, 0, 0)),
            pl.BlockSpec((1, 1, 32), lambda h, i: (h, 0, 0)),
        ],
        out_specs=pl.BlockSpec((bm, 32), lambda h, i: (h * nb + i, 0)),
        out_shape=jax.ShapeDtypeStruct((2 * N, 32), jnp.bfloat16),
    )(x, w3, b3)


def _mid_body(af_ref, df_ref, acb_ref, dcb_ref, w_ref, b_ref, o_ref):
    df = jnp.maximum(df_ref[...], 1.0)
    dcb = jnp.maximum(dcb_ref[...], 1.0)
    h_lo = (af_ref[0].astype(jnp.float32) / df
            + acb_ref[0].astype(jnp.float32) / dcb)
    h_hi = (af_ref[1].astype(jnp.float32) / df
            + acb_ref[1].astype(jnp.float32) / dcb)
    h = jnp.concatenate([h_lo, h_hi], axis=1)
    h = jnp.where(h >= 0, h, 0.01 * h)
    o_ref[...] = jnp.dot(h, w_ref[0],
                         preferred_element_type=jnp.float32) + b_ref[0]


def _mid_layer(acc_f, deg_f, acc_cb, deg_cb, w2, b2):
    bm = 2048
    nb = N_ACC // bm
    w3 = w2.reshape(HIDDEN, 2, 16).transpose(1, 0, 2)
    b3 = b2.reshape(2, 1, 16)
    return pl.pallas_call(
        _mid_body,
        grid=(2, nb),
        in_specs=[
            pl.BlockSpec((2, bm, 32), lambda h, i: (0, i, 0)),
            pl.BlockSpec((bm, 1), lambda h, i: (i, 0)),
            pl.BlockSpec((2, bm, 32), lambda h, i: (0, i, 0)),
            pl.BlockSpec((bm, 1), lambda h, i: (i, 0)),
            pl.BlockSpec((1, HIDDEN, 16), lambda h, i: (h, 0, 0)),
            pl.BlockSpec((1, 1, 16), lambda h, i: (h, 0, 0)),
        ],
        out_specs=pl.BlockSpec((bm, 16), lambda h, i: (h * nb + i, 0)),
        out_shape=jax.ShapeDtypeStruct((2 * N_ACC, 16), jnp.float32),
    )(acc_f, deg_f, acc_cb, deg_cb, w3, b3)


def _out_body(a_ref, d_ref, o_ref):
    d = jnp.maximum(d_ref[...], 1.0)
    o_ref[...] = jnp.concatenate([a_ref[0] / d, a_ref[1] / d], axis=1)


def _out_layer(acc_c, deg_c):
    bm = 2000
    nb = N // bm
    return pl.pallas_call(
        _out_body,
        grid=(nb,),
        in_specs=[
            pl.BlockSpec((2, bm, 16), lambda i: (0, i, 0)),
            pl.BlockSpec((bm, 1), lambda i: (i, 0)),
        ],
        out_specs=pl.BlockSpec((bm, OUT_SIZE), lambda i: (i, 0)),
        out_shape=jax.ShapeDtypeStruct((N, OUT_SIZE), jnp.float32),
    )(acc_c, deg_c)


def kernel(x_user, x_item,
           W1_follows, b1_follows, W1_clicks, b1_clicks, W1_cb, b1_cb,
           W2_follows, b2_follows, W2_clicks, b2_clicks, W2_cb, b2_cb,
           edge_follows, edge_clicks, edge_clickedby):
    tbl_f = _linear_split(x_user, W1_follows, b1_follows)
    tbl_cb = _linear_split(x_item, W1_cb, b1_cb)

    sc32x2 = _make_sc_pass(32, jnp.bfloat16, 2)
    acc_f, deg_f, acc_cb, deg_cb = sc32x2(
        tbl_f, *_prep_edges(edge_follows, N),
        tbl_cb, *_prep_edges(edge_clickedby, N))

    tbl_c = _mid_layer(acc_f, deg_f[:, None], acc_cb, deg_cb[:, None],
                       W2_clicks, b2_clicks)

    sc16 = _make_sc_pass(16, jnp.float32, 1)
    acc_c, deg_c = sc16(tbl_c, *_prep_edges(edge_clicks, N_ACC))

    return _out_layer(acc_c, deg_c[:N, None])


# speedup vs baseline: 7.5078x; 1.1360x over previous
import functools

import jax
import jax.numpy as jnp
from jax import lax
from jax.experimental import pallas as pl
from jax.experimental.pallas import tpu as pltpu
from jax.experimental.pallas import tpu_sc as plsc

N = 50000
E = 500000
IN_SIZE = 128
HIDDEN = 64
OUT_SIZE = 32

N_ACC = 51200
R_TILE = N_ACC // 16
CHUNK = 128
E_PAD = 503808
K_CHUNKS = E_PAD // 16 // CHUNK
DRAIN = R_TILE // CHUNK


def _sc_body(fw, dt, npass, tbl_rows, *args):
    ins = args[:3 * npass]
    outs = args[3 * npass:5 * npass]
    (srcv, dstv, gb0, gb1, gb2, zbuf, onesv, dbuf, acc, deg,
     sem0, sem1, sem2) = args[5 * npass:]
    gbufs = (gb0, gb1, gb2)
    sems = (sem0, sem1, sem2)

    c = lax.axis_index("c")
    s = lax.axis_index("s")
    zero16 = jnp.zeros((16,), jnp.float32)
    one16 = jnp.ones((16,), jnp.float32)
    lanes = 16 if dt == jnp.float32 else 32
    zerov = jnp.zeros((lanes,), dt)

    def _fill_z(r, _):
        for q in range(fw // lanes):
            zbuf[r, pl.ds(q * lanes, lanes)] = zerov
        return 0
    lax.fori_loop(0, CHUNK, _fill_z, 0)

    def _fill_o(i, _):
        onesv[pl.ds(i * 16, 16)] = one16
        return 0
    lax.fori_loop(0, CHUNK // 16, _fill_o, 0)

    def _fill_d(i, _):
        dbuf[pl.ds(i * 16, 16)] = zero16
        return 0
    lax.fori_loop(0, R_TILE // 16, _fill_d, 0)

    for p in range(npass):
        tbl, srci, dsti = ins[3 * p:3 * p + 3]
        out_acc, out_deg = outs[2 * p:2 * p + 2]
        deg_core = p % 2

        pltpu.sync_copy(srci.at[s], srcv)
        pltpu.sync_copy(dsti.at[s], dstv)
        offv = jnp.zeros((16,), jnp.int32) + c * tbl_rows[p]

        def _off(r, _):
            for q in range(CHUNK // 16):
                sl = pl.ds(q * 16, 16)
                srcv[r, sl] = srcv[r, sl] + offv
            return 0
        lax.fori_loop(0, K_CHUNKS, _off, 0)

        def _zero_acc(j, _):
            pltpu.sync_copy(zbuf, acc.at[pl.ds(s * R_TILE + j * CHUNK, CHUNK)])
            return 0
        lax.fori_loop(0, DRAIN, _zero_acc, 0)

        @pl.when(c == deg_core)
        def _():
            pltpu.sync_copy(dbuf, deg.at[pl.ds(s * R_TILE, R_TILE)])

        plsc.subcore_barrier()

        for u in range(3):
            pltpu.async_copy(tbl.at[srcv.at[u]], gbufs[u], sems[u])

        def _edge(j3, _):
            base = 3 * j3
            for u in range(3):
                j = base + u
                pltpu.make_async_copy(tbl.at[srcv.at[j]], gbufs[u],
                                      sems[u]).wait()
                pltpu.sync_copy(gbufs[u], acc.at[dstv.at[j]], add=True)

                @pl.when(c == deg_core)
                def _():
                    pltpu.sync_copy(onesv, deg.at[dstv.at[j]], add=True)

                @pl.when(j + 3 < K_CHUNKS)
                def _():
                    pltpu.async_copy(tbl.at[srcv.at[j + 3]], gbufs[u], sems[u])
            return 0
        lax.fori_loop(0, K_CHUNKS // 3, _edge, 0)

        plsc.subcore_barrier()

        def _drain(j, _):
            r0 = s * R_TILE + j * CHUNK
            pltpu.sync_copy(acc.at[pl.ds(r0, CHUNK)], gb0)
            pltpu.sync_copy(gb0, out_acc.at[c, pl.ds(r0, CHUNK)])
            return 0
        lax.fori_loop(0, DRAIN, _drain, 0)

        @pl.when(c == deg_core)
        def _():
            pltpu.sync_copy(deg.at[pl.ds(s * R_TILE, R_TILE)], dbuf)
            pltpu.sync_copy(dbuf, out_deg.at[pl.ds(s * R_TILE, R_TILE)])
            if p + 1 < npass:
                lax.fori_loop(0, R_TILE // 16, _fill_d, 0)


@functools.lru_cache(maxsize=None)
def _make_sc_pass(fw, dt, npass, tbl_rows):
    mesh = plsc.VectorSubcoreMesh(core_axis_name="c", subcore_axis_name="s")
    return pl.kernel(
        functools.partial(_sc_body, fw, dt, npass, tbl_rows),
        out_type=(jax.ShapeDtypeStruct((2, N_ACC, fw), dt),
                  jax.ShapeDtypeStruct((N_ACC,), jnp.float32)) * npass,
        mesh=mesh,
        scratch_types=[
            pltpu.VMEM((K_CHUNKS, CHUNK), jnp.int32),
            pltpu.VMEM((K_CHUNKS, CHUNK), jnp.int32),
            pltpu.VMEM((CHUNK, fw), dt),
            pltpu.VMEM((CHUNK, fw), dt),
            pltpu.VMEM((CHUNK, fw), dt),
            pltpu.VMEM((CHUNK, fw), dt),
            pltpu.VMEM((CHUNK,), jnp.float32),
            pltpu.VMEM((R_TILE,), jnp.float32),
            pltpu.VMEM_SHARED((N_ACC, fw), dt),
            pltpu.VMEM_SHARED((N_ACC,), jnp.float32),
            pltpu.SemaphoreType.DMA,
            pltpu.SemaphoreType.DMA,
            pltpu.SemaphoreType.DMA,
        ],
        compiler_params=pltpu.CompilerParams(use_tc_tiling_on_sc=False),
        name=f"rgcn_seg_sum_{fw}x{npass}",
    )


def _prep_edges(edge):
    src = edge[0].astype(jnp.int32)
    dst = edge[1].astype(jnp.int32)
    pad = E_PAD - E
    ar = jnp.arange(pad, dtype=jnp.int32)
    srci = jnp.concatenate([src, ar % N]).reshape(16, K_CHUNKS, CHUNK)
    dsti = jnp.concatenate([dst, N + (ar % (N_ACC - N))]).reshape(
        16, K_CHUNKS, CHUNK)
    return lax.optimization_barrier((srci, dsti))


def _lin_body(x_ref, w_ref, b_ref, o_ref):
    o_ref[...] = (jnp.dot(x_ref[...], w_ref[0],
                          preferred_element_type=jnp.float32)
                  + b_ref[0]).astype(jnp.bfloat16)


def _linear_split(x, w, b):
    bm = 2000
    nb = N // bm
    w3 = w.reshape(IN_SIZE, 2, 32).transpose(1, 0, 2)
    b3 = b.reshape(2, 1, 32)
    return pl.pallas_call(
        _lin_body,
        grid=(2, nb),
        in_specs=[
            pl.BlockSpec((bm, IN_SIZE), lambda h, i: (i, 0)),
            pl.BlockSpec((1, IN_SIZE, 32), lambda h, i: (h, 0, 0)),
            pl.BlockSpec((1, 1, 32), lambda h, i: (h, 0, 0)),
        ],
        out_specs=pl.BlockSpec((bm, 32), lambda h, i: (h * nb + i, 0)),
        out_shape=jax.ShapeDtypeStruct((2 * N, 32), jnp.bfloat16),
    )(x, w3, b3)


def _mid_body(af_ref, df_ref, acb_ref, dcb_ref, w_ref, b_ref, o_ref):
    df = jnp.maximum(df_ref[...], 1.0)
    dcb = jnp.maximum(dcb_ref[...], 1.0)
    h_lo = (af_ref[0].astype(jnp.float32) / df
            + acb_ref[0].astype(jnp.float32) / dcb)
    h_hi = (af_ref[1].astype(jnp.float32) / df
            + acb_ref[1].astype(jnp.float32) / dcb)
    h = jnp.concatenate([h_lo, h_hi], axis=1)
    h = jnp.where(h >= 0, h, 0.01 * h)
    o_ref[...] = jnp.dot(h, w_ref[0],
                         preferred_element_type=jnp.float32) + b_ref[0]


def _mid_layer(acc_f, deg_f, acc_cb, deg_cb, w2, b2):
    bm = 2048
    nb = N_ACC // bm
    w3 = w2.reshape(HIDDEN, 2, 16).transpose(1, 0, 2)
    b3 = b2.reshape(2, 1, 16)
    return pl.pallas_call(
        _mid_body,
        grid=(2, nb),
        in_specs=[
            pl.BlockSpec((2, bm, 32), lambda h, i: (0, i, 0)),
            pl.BlockSpec((bm, 1), lambda h, i: (i, 0)),
            pl.BlockSpec((2, bm, 32), lambda h, i: (0, i, 0)),
            pl.BlockSpec((bm, 1), lambda h, i: (i, 0)),
            pl.BlockSpec((1, HIDDEN, 16), lambda h, i: (h, 0, 0)),
            pl.BlockSpec((1, 1, 16), lambda h, i: (h, 0, 0)),
        ],
        out_specs=pl.BlockSpec((bm, 16), lambda h, i: (h * nb + i, 0)),
        out_shape=jax.ShapeDtypeStruct((2 * N_ACC, 16), jnp.float32),
    )(acc_f, deg_f, acc_cb, deg_cb, w3, b3)


def _out_body(a_ref, d_ref, o_ref):
    d = jnp.maximum(d_ref[...], 1.0)
    o_ref[...] = jnp.concatenate([a_ref[0] / d, a_ref[1] / d], axis=1)


def _out_layer(acc_c, deg_c):
    bm = 2000
    nb = N // bm
    return pl.pallas_call(
        _out_body,
        grid=(nb,),
        in_specs=[
            pl.BlockSpec((2, bm, 16), lambda i: (0, i, 0)),
            pl.BlockSpec((bm, 1), lambda i: (i, 0)),
        ],
        out_specs=pl.BlockSpec((bm, OUT_SIZE), lambda i: (i, 0)),
        out_shape=jax.ShapeDtypeStruct((N, OUT_SIZE), jnp.float32),
    )(acc_c, deg_c)


def kernel(x_user, x_item,
           W1_follows, b1_follows, W1_clicks, b1_clicks, W1_cb, b1_cb,
           W2_follows, b2_follows, W2_clicks, b2_clicks, W2_cb, b2_cb,
           edge_follows, edge_clicks, edge_clickedby):
    tbl_f = _linear_split(x_user, W1_follows, b1_follows)
    tbl_cb = _linear_split(x_item, W1_cb, b1_cb)

    sc32x2 = _make_sc_pass(32, jnp.bfloat16, 2, (N, N))
    acc_f, deg_f, acc_cb, deg_cb = sc32x2(
        tbl_f, *_prep_edges(edge_follows),
        tbl_cb, *_prep_edges(edge_clickedby))

    tbl_c = _mid_layer(acc_f, deg_f[:, None], acc_cb, deg_cb[:, None],
                       W2_clicks, b2_clicks)

    sc16 = _make_sc_pass(16, jnp.float32, 1, (N_ACC,))
    acc_c, deg_c = sc16(tbl_c, *_prep_edges(edge_clicks))

    return _out_layer(acc_c, deg_c[:N, None])


# speedup vs baseline: 7.8749x; 1.0489x over previous
import functools

import jax
import jax.numpy as jnp
from jax import lax
from jax.experimental import pallas as pl
from jax.experimental.pallas import tpu as pltpu
from jax.experimental.pallas import tpu_sc as plsc

N = 50000
E = 500000
IN_SIZE = 128
HIDDEN = 64
OUT_SIZE = 32

N_ACC = 51200
R_TILE = N_ACC // 16
CHUNK = 128
E_PAD = 503808
K_CHUNKS = E_PAD // 16 // CHUNK
DRAIN = R_TILE // CHUNK


def _sc_body(fw, dt, npass, tbl_rows, *args):
    ins = args[:3 * npass]
    outs = args[3 * npass:5 * npass]
    (srcv, dstv, gb0, gb1, gb2, zbuf, onesv, dbuf, acc, deg,
     sem0, sem1, sem2) = args[5 * npass:]
    gbufs = (gb0, gb1, gb2)
    sems = (sem0, sem1, sem2)

    c = lax.axis_index("c")
    s = lax.axis_index("s")
    zero16 = jnp.zeros((16,), jnp.float32)
    one16 = jnp.ones((16,), jnp.float32)
    lanes = 16 if dt == jnp.float32 else 32
    zerov = jnp.zeros((lanes,), dt)

    def _fill_z(r, _):
        for q in range(fw // lanes):
            zbuf[r, pl.ds(q * lanes, lanes)] = zerov
        return 0
    lax.fori_loop(0, CHUNK, _fill_z, 0)

    def _fill_o(i, _):
        onesv[pl.ds(i * 16, 16)] = one16
        return 0
    lax.fori_loop(0, CHUNK // 16, _fill_o, 0)

    def _fill_d(i, _):
        dbuf[pl.ds(i * 16, 16)] = zero16
        return 0
    lax.fori_loop(0, R_TILE // 16, _fill_d, 0)

    for p in range(npass):
        tbl, srci, dsti = ins[3 * p:3 * p + 3]
        out_acc, out_deg = outs[2 * p:2 * p + 2]
        deg_core = p % 2

        pltpu.sync_copy(srci.at[s], srcv)
        pltpu.sync_copy(dsti.at[s], dstv)
        offv = jnp.zeros((16,), jnp.int32) + c * tbl_rows[p]

        def _off(r, _):
            for q in range(CHUNK // 16):
                sl = pl.ds(q * 16, 16)
                srcv[r, sl] = srcv[r, sl] + offv
            return 0
        lax.fori_loop(0, K_CHUNKS, _off, 0)

        def _zero_acc(j, _):
            pltpu.sync_copy(zbuf, acc.at[pl.ds(s * R_TILE + j * CHUNK, CHUNK)])
            return 0
        lax.fori_loop(0, DRAIN, _zero_acc, 0)

        @pl.when(c == deg_core)
        def _():
            pltpu.sync_copy(dbuf, deg.at[pl.ds(s * R_TILE, R_TILE)])

        plsc.subcore_barrier()

        for u in range(3):
            pltpu.async_copy(tbl.at[srcv.at[u]], gbufs[u], sems[u])

        def _edge(j3, _):
            base = 3 * j3
            for u in range(3):
                j = base + u
                pltpu.make_async_copy(tbl.at[srcv.at[j]], gbufs[u],
                                      sems[u]).wait()
                pltpu.sync_copy(gbufs[u], acc.at[dstv.at[j]], add=True)

                @pl.when(c == deg_core)
                def _():
                    pltpu.sync_copy(onesv, deg.at[dstv.at[j]], add=True)

                @pl.when(j + 3 < K_CHUNKS)
                def _():
                    pltpu.async_copy(tbl.at[srcv.at[j + 3]], gbufs[u], sems[u])
            return 0
        lax.fori_loop(0, K_CHUNKS // 3, _edge, 0)

        plsc.subcore_barrier()

        def _drain(j, _):
            r0 = s * R_TILE + j * CHUNK
            pltpu.sync_copy(acc.at[pl.ds(r0, CHUNK)], gb0)
            pltpu.sync_copy(gb0, out_acc.at[c, pl.ds(r0, CHUNK)])
            return 0
        lax.fori_loop(0, DRAIN, _drain, 0)

        @pl.when(c == deg_core)
        def _():
            pltpu.sync_copy(deg.at[pl.ds(s * R_TILE, R_TILE)], dbuf)
            pltpu.sync_copy(dbuf, out_deg.at[pl.ds(s * R_TILE, R_TILE)])
            if p + 1 < npass:
                lax.fori_loop(0, R_TILE // 16, _fill_d, 0)


@functools.lru_cache(maxsize=None)
def _make_sc_pass(fw, dt, npass, tbl_rows):
    mesh = plsc.VectorSubcoreMesh(core_axis_name="c", subcore_axis_name="s")
    return pl.kernel(
        functools.partial(_sc_body, fw, dt, npass, tbl_rows),
        out_type=(jax.ShapeDtypeStruct((2, N_ACC, fw), dt),
                  jax.ShapeDtypeStruct((N_ACC,), jnp.float32)) * npass,
        mesh=mesh,
        scratch_types=[
            pltpu.VMEM((K_CHUNKS, CHUNK), jnp.int32),
            pltpu.VMEM((K_CHUNKS, CHUNK), jnp.int32),
            pltpu.VMEM((CHUNK, fw), dt),
            pltpu.VMEM((CHUNK, fw), dt),
            pltpu.VMEM((CHUNK, fw), dt),
            pltpu.VMEM((CHUNK, fw), dt),
            pltpu.VMEM((CHUNK,), jnp.float32),
            pltpu.VMEM((R_TILE,), jnp.float32),
            pltpu.VMEM_SHARED((N_ACC, fw), dt),
            pltpu.VMEM_SHARED((N_ACC,), jnp.float32),
            pltpu.SemaphoreType.DMA,
            pltpu.SemaphoreType.DMA,
            pltpu.SemaphoreType.DMA,
        ],
        compiler_params=pltpu.CompilerParams(use_tc_tiling_on_sc=False),
        name=f"rgcn_seg_sum_{fw}x{npass}",
    )


def _prep_edges(edge):
    src = edge[0].astype(jnp.int32)
    dst = edge[1].astype(jnp.int32)
    pad = E_PAD - E
    ar = jnp.arange(pad, dtype=jnp.int32)
    srci = jnp.concatenate([src, ar % N]).reshape(16, K_CHUNKS, CHUNK)
    dsti = jnp.concatenate([dst, N + (ar % (N_ACC - N))]).reshape(
        16, K_CHUNKS, CHUNK)
    return lax.optimization_barrier((srci, dsti))


def _lin_body(x_ref, w_ref, b_ref, o_ref):
    y = (jnp.dot(x_ref[...], w_ref[...],
                 preferred_element_type=jnp.float32)
         + b_ref[...]).astype(jnp.bfloat16)
    o_ref[0] = y[:, :32]
    o_ref[1] = y[:, 32:]


def _linear_split(x, w, b):
    bm = 2000
    nb = N // bm
    return pl.pallas_call(
        _lin_body,
        grid=(nb,),
        in_specs=[
            pl.BlockSpec((bm, IN_SIZE), lambda i: (i, 0)),
            pl.BlockSpec((IN_SIZE, HIDDEN), lambda i: (0, 0)),
            pl.BlockSpec((1, HIDDEN), lambda i: (0, 0)),
        ],
        out_specs=pl.BlockSpec((2, bm, 32), lambda i: (0, i, 0)),
        out_shape=jax.ShapeDtypeStruct((2, N, 32), jnp.bfloat16),
    )(x, w, b.reshape(1, HIDDEN)).reshape(2 * N, 32)


def _mid_body(af_ref, df_ref, acb_ref, dcb_ref, w_ref, b_ref, o_ref):
    df = jnp.maximum(df_ref[...], 1.0)
    dcb = jnp.maximum(dcb_ref[...], 1.0)
    h_lo = (af_ref[0].astype(jnp.float32) / df
            + acb_ref[0].astype(jnp.float32) / dcb)
    h_hi = (af_ref[1].astype(jnp.float32) / df
            + acb_ref[1].astype(jnp.float32) / dcb)
    h = jnp.concatenate([h_lo, h_hi], axis=1)
    h = jnp.where(h >= 0, h, 0.01 * h)
    o_ref[...] = jnp.dot(h, w_ref[0],
                         preferred_element_type=jnp.float32) + b_ref[0]


def _mid_layer(acc_f, deg_f, acc_cb, deg_cb, w2, b2):
    bm = 2048
    nb = N_ACC // bm
    w3 = w2.reshape(HIDDEN, 2, 16).transpose(1, 0, 2)
    b3 = b2.reshape(2, 1, 16)
    return pl.pallas_call(
        _mid_body,
        grid=(2, nb),
        in_specs=[
            pl.BlockSpec((2, bm, 32), lambda h, i: (0, i, 0)),
            pl.BlockSpec((bm, 1), lambda h, i: (i, 0)),
            pl.BlockSpec((2, bm, 32), lambda h, i: (0, i, 0)),
            pl.BlockSpec((bm, 1), lambda h, i: (i, 0)),
            pl.BlockSpec((1, HIDDEN, 16), lambda h, i: (h, 0, 0)),
            pl.BlockSpec((1, 1, 16), lambda h, i: (h, 0, 0)),
        ],
        out_specs=pl.BlockSpec((bm, 16), lambda h, i: (h * nb + i, 0)),
        out_shape=jax.ShapeDtypeStruct((2 * N_ACC, 16), jnp.float32),
    )(acc_f, deg_f, acc_cb, deg_cb, w3, b3)


def _out_body(a_ref, d_ref, o_ref):
    d = jnp.maximum(d_ref[...], 1.0)
    o_ref[...] = jnp.concatenate([a_ref[0] / d, a_ref[1] / d], axis=1)


def _out_layer(acc_c, deg_c):
    bm = 2000
    nb = N // bm
    return pl.pallas_call(
        _out_body,
        grid=(nb,),
        in_specs=[
            pl.BlockSpec((2, bm, 16), lambda i: (0, i, 0)),
            pl.BlockSpec((bm, 1), lambda i: (i, 0)),
        ],
        out_specs=pl.BlockSpec((bm, OUT_SIZE), lambda i: (i, 0)),
        out_shape=jax.ShapeDtypeStruct((N, OUT_SIZE), jnp.float32),
    )(acc_c, deg_c)


def kernel(x_user, x_item,
           W1_follows, b1_follows, W1_clicks, b1_clicks, W1_cb, b1_cb,
           W2_follows, b2_follows, W2_clicks, b2_clicks, W2_cb, b2_cb,
           edge_follows, edge_clicks, edge_clickedby):
    tbl_f = _linear_split(x_user, W1_follows, b1_follows)
    tbl_cb = _linear_split(x_item, W1_cb, b1_cb)

    sc32x2 = _make_sc_pass(32, jnp.bfloat16, 2, (N, N))
    acc_f, deg_f, acc_cb, deg_cb = sc32x2(
        tbl_f, *_prep_edges(edge_follows),
        tbl_cb, *_prep_edges(edge_clickedby))

    tbl_c = _mid_layer(acc_f, deg_f[:, None], acc_cb, deg_cb[:, None],
                       W2_clicks, b2_clicks)

    sc16 = _make_sc_pass(16, jnp.float32, 1, (N_ACC,))
    acc_c, deg_c = sc16(tbl_c, *_prep_edges(edge_clicks))

    return _out_layer(acc_c, deg_c[:, None])


# speedup vs baseline: 8.8330x; 1.1217x over previous
import functools

import jax
import jax.numpy as jnp
from jax import lax
from jax.experimental import pallas as pl
from jax.experimental.pallas import tpu as pltpu
from jax.experimental.pallas import tpu_sc as plsc

N = 50000
E = 500000
IN_SIZE = 128
HIDDEN = 64
OUT_SIZE = 32

N_ACC = 51200
R_TILE = N_ACC // 16
CHUNK = 128
E_PAD = 503808
K_CHUNKS = E_PAD // 16 // CHUNK
DRAIN = R_TILE // CHUNK


def _sc_body(fw, dt, npass, tbl_rows, *args):
    ins = args[:3 * npass]
    outs = args[3 * npass:5 * npass]
    (srcv, dstv, gb0, gb1, gb2, zbuf, onesv, dbuf, acc, deg,
     sem0, sem1, sem2) = args[5 * npass:]
    gbufs = (gb0, gb1, gb2)
    sems = (sem0, sem1, sem2)

    c = lax.axis_index("c")
    s = lax.axis_index("s")
    zero16 = jnp.zeros((16,), jnp.float32)
    one16 = jnp.ones((16,), jnp.float32)
    lanes = 16 if dt == jnp.float32 else 32
    zerov = jnp.zeros((lanes,), dt)

    def _fill_z(r, _):
        for q in range(fw // lanes):
            zbuf[r, pl.ds(q * lanes, lanes)] = zerov
        return 0
    lax.fori_loop(0, CHUNK, _fill_z, 0)

    def _fill_o(i, _):
        onesv[pl.ds(i * 16, 16)] = one16
        return 0
    lax.fori_loop(0, CHUNK // 16, _fill_o, 0)

    def _fill_d(i, _):
        dbuf[pl.ds(i * 16, 16)] = zero16
        return 0
    lax.fori_loop(0, R_TILE // 16, _fill_d, 0)

    for p in range(npass):
        tbl, srci, dsti = ins[3 * p:3 * p + 3]
        out_acc, out_deg = outs[2 * p:2 * p + 2]
        deg_core = p % 2

        pltpu.sync_copy(srci.at[s], srcv)
        pltpu.sync_copy(dsti.at[s], dstv)
        offv = jnp.zeros((16,), jnp.int32) + c * tbl_rows[p]

        def _off(r, _):
            for q in range(CHUNK // 16):
                sl = pl.ds(q * 16, 16)
                srcv[r, sl] = srcv[r, sl] + offv
            return 0
        lax.fori_loop(0, K_CHUNKS, _off, 0)

        def _zero_acc(j, _):
            pltpu.sync_copy(zbuf, acc.at[pl.ds(s * R_TILE + j * CHUNK, CHUNK)])
            return 0
        lax.fori_loop(0, DRAIN, _zero_acc, 0)

        @pl.when(c == deg_core)
        def _():
            pltpu.sync_copy(dbuf, deg.at[pl.ds(s * R_TILE, R_TILE)])

        plsc.subcore_barrier()

        for u in range(3):
            pltpu.async_copy(tbl.at[srcv.at[u]], gbufs[u], sems[u])

        def _edge(j3, _):
            base = 3 * j3
            for u in range(3):
                j = base + u
                pltpu.make_async_copy(tbl.at[srcv.at[j]], gbufs[u],
                                      sems[u]).wait()
                pltpu.sync_copy(gbufs[u], acc.at[dstv.at[j]], add=True)

                @pl.when(c == deg_core)
                def _():
                    pltpu.sync_copy(onesv, deg.at[dstv.at[j]], add=True)

                @pl.when(j + 3 < K_CHUNKS)
                def _():
                    pltpu.async_copy(tbl.at[srcv.at[j + 3]], gbufs[u], sems[u])
            return 0
        lax.fori_loop(0, K_CHUNKS // 3, _edge, 0)

        plsc.subcore_barrier()

        def _drain(j, _):
            r0 = s * R_TILE + j * CHUNK
            pltpu.sync_copy(acc.at[pl.ds(r0, CHUNK)], gb0)
            pltpu.sync_copy(gb0, out_acc.at[c, pl.ds(r0, CHUNK)])
            return 0
        lax.fori_loop(0, DRAIN, _drain, 0)

        @pl.when(c == deg_core)
        def _():
            pltpu.sync_copy(deg.at[pl.ds(s * R_TILE, R_TILE)], dbuf)
            pltpu.sync_copy(dbuf, out_deg.at[pl.ds(s * R_TILE, R_TILE)])
            if p + 1 < npass:
                lax.fori_loop(0, R_TILE // 16, _fill_d, 0)


@functools.lru_cache(maxsize=None)
def _make_sc_pass(fw, dt, npass, tbl_rows):
    mesh = plsc.VectorSubcoreMesh(core_axis_name="c", subcore_axis_name="s")
    return pl.kernel(
        functools.partial(_sc_body, fw, dt, npass, tbl_rows),
        out_type=(jax.ShapeDtypeStruct((2, N_ACC, fw), dt),
                  jax.ShapeDtypeStruct((N_ACC,), jnp.float32)) * npass,
        mesh=mesh,
        scratch_types=[
            pltpu.VMEM((K_CHUNKS, CHUNK), jnp.int32),
            pltpu.VMEM((K_CHUNKS, CHUNK), jnp.int32),
            pltpu.VMEM((CHUNK, fw), dt),
            pltpu.VMEM((CHUNK, fw), dt),
            pltpu.VMEM((CHUNK, fw), dt),
            pltpu.VMEM((CHUNK, fw), dt),
            pltpu.VMEM((CHUNK,), jnp.float32),
            pltpu.VMEM((R_TILE,), jnp.float32),
            pltpu.VMEM_SHARED((N_ACC, fw), dt),
            pltpu.VMEM_SHARED((N_ACC,), jnp.float32),
            pltpu.SemaphoreType.DMA,
            pltpu.SemaphoreType.DMA,
            pltpu.SemaphoreType.DMA,
        ],
        compiler_params=pltpu.CompilerParams(use_tc_tiling_on_sc=False),
        name=f"rgcn_seg_sum_{fw}x{npass}",
    )


def _prep_edges(edge):
    src = edge[0].astype(jnp.int32)
    dst = edge[1].astype(jnp.int32)
    pad = E_PAD - E
    ar = jnp.arange(pad, dtype=jnp.int32)
    srci = jnp.concatenate([src, ar % N]).reshape(16, K_CHUNKS, CHUNK)
    dsti = jnp.concatenate([dst, N + (ar % (N_ACC - N))]).reshape(
        16, K_CHUNKS, CHUNK)
    return lax.optimization_barrier((srci, dsti))


def _lin_body(x_ref, w_ref, b_ref, o_ref):
    y = (jnp.dot(x_ref[...], w_ref[...],
                 preferred_element_type=jnp.float32)
         + b_ref[...]).astype(jnp.bfloat16)
    o_ref[0] = y[:, :32]
    o_ref[1] = y[:, 32:]


def _linear_split(x, w, b):
    bm = 2000
    nb = N // bm
    return pl.pallas_call(
        _lin_body,
        grid=(nb,),
        in_specs=[
            pl.BlockSpec((bm, IN_SIZE), lambda i: (i, 0)),
            pl.BlockSpec((IN_SIZE, HIDDEN), lambda i: (0, 0)),
            pl.BlockSpec((1, HIDDEN), lambda i: (0, 0)),
        ],
        out_specs=pl.BlockSpec((2, bm, 32), lambda i: (0, i, 0)),
        out_shape=jax.ShapeDtypeStruct((2, N, 32), jnp.bfloat16),
    )(x, w, b.reshape(1, HIDDEN)).reshape(2 * N, 32)


def _mid_body(af_ref, df_ref, acb_ref, dcb_ref, a_ref, bb_ref, bx_ref, o_ref):
    df = jnp.maximum(df_ref[...], 1.0)
    dcb = jnp.maximum(dcb_ref[...], 1.0)
    h_lo = (af_ref[0].astype(jnp.float32) / df
            + acb_ref[0].astype(jnp.float32) / dcb)
    h_hi = (af_ref[1].astype(jnp.float32) / df
            + acb_ref[1].astype(jnp.float32) / dcb)
    h_lo = jnp.where(h_lo >= 0, h_lo, 0.01 * h_lo)
    h_hi = jnp.where(h_hi >= 0, h_hi, 0.01 * h_hi)
    o_ref[0] = (jnp.dot(h_lo, a_ref[0], preferred_element_type=jnp.float32)
                + jnp.dot(h_hi, bb_ref[0], preferred_element_type=jnp.float32)
                + bx_ref[0])


def _mid_layer(acc_f, deg_f, acc_cb, deg_cb, w2, b2):
    rp = N_ACC // 4
    af = acc_f.reshape(2, rp, 128)
    acb = acc_cb.reshape(2, rp, 128)
    dfx = jnp.repeat(deg_f.reshape(rp, 4), 32, axis=1)
    dcbx = jnp.repeat(deg_cb.reshape(rp, 4), 32, axis=1)
    eye4 = jnp.eye(4, dtype=jnp.float32)
    a_bd = jnp.stack([jnp.kron(eye4, w2[:32, 16 * h:16 * h + 16])
                      for h in (0, 1)])
    b_bd = jnp.stack([jnp.kron(eye4, w2[32:, 16 * h:16 * h + 16])
                      for h in (0, 1)])
    bx = jnp.stack([jnp.tile(b2[16 * h:16 * h + 16], 4) for h in (0, 1)])
    bx = bx.reshape(2, 1, 64)
    bm = 1600
    nb = rp // bm
    out = pl.pallas_call(
        _mid_body,
        grid=(2, nb),
        in_specs=[
            pl.BlockSpec((2, bm, 128), lambda h, i: (0, i, 0)),
            pl.BlockSpec((bm, 128), lambda h, i: (i, 0)),
            pl.BlockSpec((2, bm, 128), lambda h, i: (0, i, 0)),
            pl.BlockSpec((bm, 128), lambda h, i: (i, 0)),
            pl.BlockSpec((1, 128, 64), lambda h, i: (h, 0, 0)),
            pl.BlockSpec((1, 128, 64), lambda h, i: (h, 0, 0)),
            pl.BlockSpec((1, 1, 64), lambda h, i: (h, 0, 0)),
        ],
        out_specs=pl.BlockSpec((1, bm, 64), lambda h, i: (h, i, 0)),
        out_shape=jax.ShapeDtypeStruct((2, rp, 64), jnp.float32),
    )(af, dfx, acb, dcbx, a_bd, b_bd, bx)
    return out.reshape(2 * N_ACC, 16)


def _out_body(a_ref, d_ref, o_ref):
    d = jnp.maximum(d_ref[...], 1.0)
    o_ref[...] = jnp.concatenate([a_ref[0] / d, a_ref[1] / d], axis=1)


def _out_layer(acc_c, deg_c):
    bm = 2000
    nb = N // bm
    return pl.pallas_call(
        _out_body,
        grid=(nb,),
        in_specs=[
            pl.BlockSpec((2, bm, 16), lambda i: (0, i, 0)),
            pl.BlockSpec((bm, 1), lambda i: (i, 0)),
        ],
        out_specs=pl.BlockSpec((bm, OUT_SIZE), lambda i: (i, 0)),
        out_shape=jax.ShapeDtypeStruct((N, OUT_SIZE), jnp.float32),
    )(acc_c, deg_c)


def kernel(x_user, x_item,
           W1_follows, b1_follows, W1_clicks, b1_clicks, W1_cb, b1_cb,
           W2_follows, b2_follows, W2_clicks, b2_clicks, W2_cb, b2_cb,
           edge_follows, edge_clicks, edge_clickedby):
    tbl_f = _linear_split(x_user, W1_follows, b1_follows)
    tbl_cb = _linear_split(x_item, W1_cb, b1_cb)

    sc32x2 = _make_sc_pass(32, jnp.bfloat16, 2, (N, N))
    acc_f, deg_f, acc_cb, deg_cb = sc32x2(
        tbl_f, *_prep_edges(edge_follows),
        tbl_cb, *_prep_edges(edge_clickedby))

    tbl_c = _mid_layer(acc_f, deg_f, acc_cb, deg_cb, W2_clicks, b2_clicks)

    sc16 = _make_sc_pass(16, jnp.float32, 1, (N_ACC,))
    acc_c, deg_c = sc16(tbl_c, *_prep_edges(edge_clicks))

    return _out_layer(acc_c, deg_c[:, None])
